# Initial kernel scaffold; baseline (speedup 1.0000x reference)
import jax
import jax.numpy as jnp
from jax.experimental import pallas as pl


def kernel(x, spatial_edge_index, gene_sim_edge_index, W_sp, a_src_sp, a_dst_sp, b_sp, W_gs, a_src_gs, a_dst_gs, b_gs, W_f, a_src_f, a_dst_f, b_f):
    raise NotImplementedError("write your pallas kernel here")


# speedup vs baseline: 15.0425x; 15.0425x over previous
import dataclasses
import functools

import jax
import jax.numpy as jnp
from jax import lax
from jax.experimental import pallas as pl
from jax.experimental.pallas import tpu as pltpu
from jax.experimental.pallas import tpu_sc as plsc

N = 10000
D = 128
DH = 64
E = 320000
NEG_SLOPE = 0.2

NSUB = 16
NCORE = 2
B = 80
EDGES_PER_TILE = E // NSUB
NCHUNK = EDGES_PER_TILE // B
NPAD = 10240
ROWS_PER_TILE = NPAD // NSUB
ROWS_LAST = N - 15 * ROWS_PER_TILE


def _dense_in_body(x_ref, W_ref, asrc_ref, adst_ref, h2_ref, as_ref, ad_ref):
    h = jnp.dot(x_ref[...], W_ref[...], preferred_element_type=jnp.float32)
    h2_ref[0] = h[:, :DH]
    h2_ref[1] = h[:, DH:]
    as_ref[...] = jnp.sum(h * asrc_ref[...][None, :], axis=1, keepdims=True)
    ad_ref[...] = jnp.sum(h * adst_ref[...][None, :], axis=1, keepdims=True)


def _dense_in_fused_body(o1_ref, o2_ref, W_ref, asrc_ref, adst_ref,
                         h2_ref, as_ref, ad_ref):
    xv = (o1_ref[...] + o2_ref[...]) * 0.5
    h = jnp.dot(xv, W_ref[...], preferred_element_type=jnp.float32)
    h2_ref[0] = h[:, :DH]
    h2_ref[1] = h[:, DH:]
    as_ref[...] = jnp.sum(h * asrc_ref[...][None, :], axis=1, keepdims=True)
    ad_ref[...] = jnp.sum(h * adst_ref[...][None, :], axis=1, keepdims=True)


_DENSE_IN_OUT = [
    jax.ShapeDtypeStruct((NCORE, N, DH), jnp.float32),
    jax.ShapeDtypeStruct((N, 1), jnp.float32),
    jax.ShapeDtypeStruct((N, 1), jnp.float32),
]


def _dense_in(x, W, asrc, adst):
    return pl.pallas_call(_dense_in_body, out_shape=_DENSE_IN_OUT)(
        x, W, asrc, adst)


def _dense_in_fused(o1, o2, W, asrc, adst):
    return pl.pallas_call(_dense_in_fused_body, out_shape=_DENSE_IN_OUT)(
        o1, o2, W, asrc, adst)


def _dense_out_body(num_ref, den_ref, h2_ref, as_ref, ad_ref, b_ref, o_ref,
                    *, use_elu):
    a = as_ref[...] + ad_ref[...]
    a = jnp.where(a >= 0.0, a, a * NEG_SLOPE)
    ws = jnp.exp(a)
    inv = 1.0 / (den_ref[...] + ws + 1e-16)
    b = b_ref[...]
    for c in range(NCORE):
        o = (num_ref[c] + ws * h2_ref[c]) * inv + b[c * DH:(c + 1) * DH][None, :]
        if use_elu:
            o = jnp.where(o > 0.0, o, jnp.exp(o) - 1.0)
        o_ref[:, c * DH:(c + 1) * DH] = o


def _dense_out(num, den, h2, a_s, a_d, bias, use_elu):
    body = functools.partial(_dense_out_body, use_elu=use_elu)
    return pl.pallas_call(
        body, out_shape=jax.ShapeDtypeStruct((N, D), jnp.float32),
    )(num, den, h2, a_s, a_d, bias)


def _sc_agg_body(h2f_hbm, as_hbm, ad_hbm, src_hbm, dst_hbm,
                 numf_hbm, den_hbm,
                 src_t, dst_t, idx_t, w_t, rows_t, as_t, ad_t,
                 num_sp, den_sp, sem):
    cid = lax.axis_index("c")
    sid = lax.axis_index("s")

    zero16 = jnp.zeros((16,), jnp.float32)

    pltpu.sync_copy(as_hbm, as_t)
    pltpu.sync_copy(ad_hbm, ad_t)

    @pl.loop(0, B)
    def _zrow(r):
        for f in range(4):
            rows_t[r, pl.ds(16 * f, 16)] = zero16

    for j in range(B // 16):
        w_t[pl.ds(16 * j, 16)] = zero16

    row0 = sid * ROWS_PER_TILE
    for k in range(ROWS_PER_TILE // B):
        pltpu.sync_copy(rows_t, num_sp.at[pl.ds(row0 + k * B, B)])
        pltpu.sync_copy(w_t, den_sp.at[pl.ds(row0 + k * B, B)])
    plsc.subcore_barrier()

    coff = jnp.full((16,), cid * N, jnp.int32)

    @pl.loop(0, NCHUNK)
    def _chunk(c):
        base = sid * EDGES_PER_TILE + c * B
        pltpu.sync_copy(src_hbm.at[pl.ds(base, B)], src_t)
        pltpu.sync_copy(dst_hbm.at[pl.ds(base, B)], dst_t)

        for j in range(B // 16):
            idx_t[pl.ds(16 * j, 16)] = src_t[pl.ds(16 * j, 16)] + coff

        gcp = pltpu.async_copy(h2f_hbm.at[idx_t], rows_t, sem)

        for j in range(B // 16):
            sv = src_t[pl.ds(16 * j, 16)]
            dv = dst_t[pl.ds(16 * j, 16)]
            e = plsc.load_gather(as_t, [sv]) + plsc.load_gather(ad_t, [dv])
            e = jnp.where(e >= 0.0, e, e * NEG_SLOPE)
            w_t[pl.ds(16 * j, 16)] = jnp.exp(e)

        gcp.wait()

        @pl.loop(0, B)
        def _row(r):
            wr = plsc.load_gather(w_t, [jnp.full((16,), 0, jnp.int32) + r])
            for f in range(4):
                sl = pl.ds(16 * f, 16)
                rows_t[r, sl] = rows_t[r, sl] * wr

        pltpu.sync_copy(rows_t, num_sp.at[dst_t], add=True)

        @pl.when(cid == 0)
        def _den():
            pltpu.sync_copy(w_t, den_sp.at[dst_t], add=True)

    plsc.subcore_barrier()

    out0 = cid * N + row0

    @pl.when(sid < NSUB - 1)
    def _wfull():
        pltpu.sync_copy(num_sp.at[pl.ds(row0, ROWS_PER_TILE)],
                        numf_hbm.at[pl.ds(out0, ROWS_PER_TILE)])

    @pl.when(sid == NSUB - 1)
    def _wlast():
        pltpu.sync_copy(num_sp.at[pl.ds(row0, ROWS_LAST)],
                        numf_hbm.at[pl.ds(out0, ROWS_LAST)])

    @pl.when(cid == 0)
    def _wden():
        @pl.when(sid < NSUB - 1)
        def _dfull():
            pltpu.sync_copy(den_sp.at[pl.ds(row0, ROWS_PER_TILE)],
                            den_hbm.at[pl.ds(row0, ROWS_PER_TILE)])

        @pl.when(sid == NSUB - 1)
        def _dlast():
            pltpu.sync_copy(den_sp.at[pl.ds(row0, ROWS_LAST)],
                            den_hbm.at[pl.ds(row0, ROWS_LAST)])


def _sc_agg(h2, a_s, a_d, src, dst):
    mesh = plsc.VectorSubcoreMesh(core_axis_name="c", subcore_axis_name="s")
    h2f = h2.reshape(NCORE * N, DH)
    cp = pltpu.CompilerParams()
    if "needs_layout_passes" in pltpu.CompilerParams.__dataclass_fields__:
        cp = dataclasses.replace(cp, needs_layout_passes=False)
    if "use_tc_tiling_on_sc" in pltpu.CompilerParams.__dataclass_fields__:
        cp = dataclasses.replace(cp, use_tc_tiling_on_sc=False)
    kern = pl.kernel(
        _sc_agg_body,
        out_type=[
            jax.ShapeDtypeStruct((NCORE * N, DH), jnp.float32),
            jax.ShapeDtypeStruct((N,), jnp.float32),
        ],
        mesh=mesh,
        scratch_types=[
            pltpu.VMEM((B,), jnp.int32),
            pltpu.VMEM((B,), jnp.int32),
            pltpu.VMEM((B,), jnp.int32),
            pltpu.VMEM((B,), jnp.float32),
            pltpu.VMEM((B, DH), jnp.float32),
            pltpu.VMEM((N,), jnp.float32),
            pltpu.VMEM((N,), jnp.float32),
            pltpu.VMEM_SHARED((NPAD, DH), jnp.float32),
            pltpu.VMEM_SHARED((NPAD,), jnp.float32),
            pltpu.SemaphoreType.DMA,
        ],
        compiler_params=cp,
    )
    return kern(h2f, a_s, a_d, src, dst)


def _gat_layer(h2, a_s, a_d, src, dst, bias, use_elu):
    num, den = _sc_agg(h2, a_s.reshape(N), a_d.reshape(N), src, dst)
    num = num.reshape(NCORE, N, DH)
    den = den.reshape(N, 1)
    return _dense_out(num, den, h2, a_s, a_d, bias, use_elu)


def kernel(x, spatial_edge_index, gene_sim_edge_index,
           W_sp, a_src_sp, a_dst_sp, b_sp,
           W_gs, a_src_gs, a_dst_gs, b_gs,
           W_f, a_src_f, a_dst_f, b_f):
    sp_src = spatial_edge_index[0]
    sp_dst = spatial_edge_index[1]
    gs_src = gene_sim_edge_index[0]
    gs_dst = gene_sim_edge_index[1]

    h2_sp, as_sp, ad_sp = _dense_in(x, W_sp, a_src_sp, a_dst_sp)
    h2_gs, as_gs, ad_gs = _dense_in(x, W_gs, a_src_gs, a_dst_gs)

    o1 = _gat_layer(h2_sp, as_sp, ad_sp, sp_src, sp_dst, b_sp, True)
    o2 = _gat_layer(h2_gs, as_gs, ad_gs, gs_src, gs_dst, b_gs, True)

    h2_f, as_f, ad_f = _dense_in_fused(o1, o2, W_f, a_src_f, a_dst_f)
    return _gat_layer(h2_f, as_f, ad_f, sp_src, sp_dst, b_f, False)


# speedup vs baseline: 49.0634x; 3.2616x over previous
import dataclasses
import functools

import jax
import jax.numpy as jnp
from jax import lax
from jax.experimental import pallas as pl
from jax.experimental.pallas import tpu as pltpu
from jax.experimental.pallas import tpu_sc as plsc

N = 10000
D = 128
DH = 64
E = 320000
NEG_SLOPE = 0.2

NSUB = 16
NCORE = 2
B = 80
EDGES_PER_TILE = E // NSUB
NCHUNK = EDGES_PER_TILE // B
NPAD = 10240
ROWS_PER_TILE = NPAD // NSUB
ROWS_LAST = N - 15 * ROWS_PER_TILE


def _dense_in_body(x_ref, W_ref, asrc_ref, adst_ref, h2_ref, as_ref, ad_ref):
    h = jnp.dot(x_ref[...], W_ref[...], preferred_element_type=jnp.float32)
    h2_ref[0] = h[:, :DH]
    h2_ref[1] = h[:, DH:]
    as_ref[...] = jnp.sum(h * asrc_ref[...][None, :], axis=1, keepdims=True)
    ad_ref[...] = jnp.sum(h * adst_ref[...][None, :], axis=1, keepdims=True)


def _dense_in_fused_body(o1_ref, o2_ref, W_ref, asrc_ref, adst_ref,
                         h2_ref, as_ref, ad_ref):
    xv = (o1_ref[...] + o2_ref[...]) * 0.5
    h = jnp.dot(xv, W_ref[...], preferred_element_type=jnp.float32)
    h2_ref[0] = h[:, :DH]
    h2_ref[1] = h[:, DH:]
    as_ref[...] = jnp.sum(h * asrc_ref[...][None, :], axis=1, keepdims=True)
    ad_ref[...] = jnp.sum(h * adst_ref[...][None, :], axis=1, keepdims=True)


_DENSE_IN_OUT = [
    jax.ShapeDtypeStruct((NCORE, N, DH), jnp.float32),
    jax.ShapeDtypeStruct((N, 1), jnp.float32),
    jax.ShapeDtypeStruct((N, 1), jnp.float32),
]


def _dense_in(x, W, asrc, adst):
    return pl.pallas_call(_dense_in_body, out_shape=_DENSE_IN_OUT)(
        x, W, asrc, adst)


def _dense_in_fused(o1, o2, W, asrc, adst):
    return pl.pallas_call(_dense_in_fused_body, out_shape=_DENSE_IN_OUT)(
        o1, o2, W, asrc, adst)


def _dense_out_body(num_ref, den_ref, h2_ref, as_ref, ad_ref, b_ref, o_ref,
                    *, use_elu):
    a = as_ref[...] + ad_ref[...]
    a = jnp.where(a >= 0.0, a, a * NEG_SLOPE)
    ws = jnp.exp(a)
    inv = 1.0 / (den_ref[...] + ws + 1e-16)
    b = b_ref[...]
    for c in range(NCORE):
        o = (num_ref[c] + ws * h2_ref[c]) * inv + b[c * DH:(c + 1) * DH][None, :]
        if use_elu:
            o = jnp.where(o > 0.0, o, jnp.exp(o) - 1.0)
        o_ref[:, c * DH:(c + 1) * DH] = o


def _dense_out(num, den, h2, a_s, a_d, bias, use_elu):
    body = functools.partial(_dense_out_body, use_elu=use_elu)
    return pl.pallas_call(
        body, out_shape=jax.ShapeDtypeStruct((N, D), jnp.float32),
    )(num, den, h2, a_s, a_d, bias)


def _sc_agg_body(h2f_hbm, as_hbm, ad_hbm, src_hbm, dst_hbm,
                 numf_hbm, den_hbm,
                 src_all, dst_all, as_t, ad_t,
                 rows_g0, rows_g1, rows_s0, rows_s1,
                 idxg0, idxg1, idxs0, idxs1,
                 wb0, wb1, wsc0, wsc1,
                 num_sp, den_sp,
                 gsem0, gsem1, ssem0, ssem1, dsem0, dsem1):
    cid = lax.axis_index("c")
    sid = lax.axis_index("s")

    rows_g = (rows_g0, rows_g1)
    rows_s = (rows_s0, rows_s1)
    idxg = (idxg0, idxg1)
    idxs = (idxs0, idxs1)
    wb = (wb0, wb1)
    wsc = (wsc0, wsc1)
    gsem = (gsem0, gsem1)
    ssem = (ssem0, ssem1)
    dsem = (dsem0, dsem1)

    zero16 = jnp.zeros((16,), jnp.float32)
    nvec = B // 16

    ebase = sid * EDGES_PER_TILE
    pltpu.sync_copy(src_hbm.at[pl.ds(ebase, EDGES_PER_TILE)], src_all)
    pltpu.sync_copy(dst_hbm.at[pl.ds(ebase, EDGES_PER_TILE)], dst_all)
    pltpu.sync_copy(as_hbm, as_t)
    pltpu.sync_copy(ad_hbm, ad_t)

    @pl.loop(0, B)
    def _zrow(r):
        for f in range(4):
            rows_g0[r, pl.ds(16 * f, 16)] = zero16

    for j in range(nvec):
        wb0[pl.ds(16 * j, 16)] = zero16

    row0 = sid * ROWS_PER_TILE
    for k in range(ROWS_PER_TILE // B):
        pltpu.sync_copy(rows_g0, num_sp.at[pl.ds(row0 + k * B, B)])
        pltpu.sync_copy(wb0, den_sp.at[pl.ds(row0 + k * B, B)])
    plsc.subcore_barrier()

    coff = jnp.full((16,), cid * N, jnp.int32)

    def prep(b, c):
        for j in range(nvec):
            sl = pl.ds(c * B + 16 * j, 16)
            sv = src_all[sl]
            dv = dst_all[sl]
            idxg[b][pl.ds(16 * j, 16)] = sv + coff
            e = plsc.load_gather(as_t, [sv]) + plsc.load_gather(ad_t, [dv])
            e = jnp.where(e >= 0.0, e, e * NEG_SLOPE)
            wb[b][pl.ds(16 * j, 16)] = jnp.exp(e)
        pltpu.async_copy(h2f_hbm.at[idxg[b]], rows_g[b], gsem[b])

    def wait_gather(b):
        pltpu.make_async_copy(h2f_hbm.at[idxg[b]], rows_g[b], gsem[b]).wait()

    def wait_scatter(b):
        pltpu.make_async_copy(rows_s[b], num_sp.at[idxs[b]], ssem[b]).wait()

    def wait_den(b):
        @pl.when(cid == 0)
        def _():
            pltpu.make_async_copy(wsc[b], den_sp.at[idxs[b]], dsem[b]).wait()

    def process(b, c):
        for j in range(nvec):
            idxs[b][pl.ds(16 * j, 16)] = dst_all[pl.ds(c * B + 16 * j, 16)]
        for j in range(nvec):
            wv = wb[b][pl.ds(16 * j, 16)]
            for r in range(16):
                wr = jnp.full((16,), wv[r])
                row = 16 * j + r
                for f in range(4):
                    sl = pl.ds(16 * f, 16)
                    rows_s[b][row, sl] = rows_g[b][row, sl] * wr
        pltpu.async_copy(rows_s[b], num_sp.at[idxs[b]], ssem[b], add=True)

        @pl.when(cid == 0)
        def _():
            for j in range(nvec):
                sl = pl.ds(16 * j, 16)
                wsc[b][sl] = wb[b][sl]
            pltpu.async_copy(wsc[b], den_sp.at[idxs[b]], dsem[b], add=True)

    prep(0, 0)
    prep(1, 1)

    for b in range(2):
        wait_gather(b)
        process(b, b)
        prep(b, b + 2)

    @pl.loop(1, NCHUNK // 2 - 1)
    def _grp(k):
        for b in range(2):
            c = 2 * k + b
            wait_scatter(b)
            wait_den(b)
            wait_gather(b)
            process(b, c)
            prep(b, c + 2)

    for b in range(2):
        c = NCHUNK - 2 + b
        wait_scatter(b)
        wait_den(b)
        wait_gather(b)
        process(b, c)

    for b in range(2):
        wait_scatter(b)
        wait_den(b)

    plsc.subcore_barrier()

    out0 = cid * N + row0

    @pl.when(sid < NSUB - 1)
    def _wfull():
        pltpu.sync_copy(num_sp.at[pl.ds(row0, ROWS_PER_TILE)],
                        numf_hbm.at[pl.ds(out0, ROWS_PER_TILE)])

    @pl.when(sid == NSUB - 1)
    def _wlast():
        pltpu.sync_copy(num_sp.at[pl.ds(row0, ROWS_LAST)],
                        numf_hbm.at[pl.ds(out0, ROWS_LAST)])

    @pl.when(cid == 0)
    def _wden():
        @pl.when(sid < NSUB - 1)
        def _dfull():
            pltpu.sync_copy(den_sp.at[pl.ds(row0, ROWS_PER_TILE)],
                            den_hbm.at[pl.ds(row0, ROWS_PER_TILE)])

        @pl.when(sid == NSUB - 1)
        def _dlast():
            pltpu.sync_copy(den_sp.at[pl.ds(row0, ROWS_LAST)],
                            den_hbm.at[pl.ds(row0, ROWS_LAST)])


def _sc_agg(h2, a_s, a_d, src, dst):
    mesh = plsc.VectorSubcoreMesh(core_axis_name="c", subcore_axis_name="s")
    h2f = h2.reshape(NCORE * N, DH)
    cp = pltpu.CompilerParams()
    if "needs_layout_passes" in pltpu.CompilerParams.__dataclass_fields__:
        cp = dataclasses.replace(cp, needs_layout_passes=False)
    if "use_tc_tiling_on_sc" in pltpu.CompilerParams.__dataclass_fields__:
        cp = dataclasses.replace(cp, use_tc_tiling_on_sc=False)
    kern = pl.kernel(
        _sc_agg_body,
        out_type=[
            jax.ShapeDtypeStruct((NCORE * N, DH), jnp.float32),
            jax.ShapeDtypeStruct((N,), jnp.float32),
        ],
        mesh=mesh,
        scratch_types=[
            pltpu.VMEM((EDGES_PER_TILE,), jnp.int32),
            pltpu.VMEM((EDGES_PER_TILE,), jnp.int32),
            pltpu.VMEM((N,), jnp.float32),
            pltpu.VMEM((N,), jnp.float32),
            pltpu.VMEM((B, DH), jnp.float32),
            pltpu.VMEM((B, DH), jnp.float32),
            pltpu.VMEM((B, DH), jnp.float32),
            pltpu.VMEM((B, DH), jnp.float32),
            pltpu.VMEM((B,), jnp.int32),
            pltpu.VMEM((B,), jnp.int32),
            pltpu.VMEM((B,), jnp.int32),
            pltpu.VMEM((B,), jnp.int32),
            pltpu.VMEM((B,), jnp.float32),
            pltpu.VMEM((B,), jnp.float32),
            pltpu.VMEM((B,), jnp.float32),
            pltpu.VMEM((B,), jnp.float32),
            pltpu.VMEM_SHARED((NPAD, DH), jnp.float32),
            pltpu.VMEM_SHARED((NPAD,), jnp.float32),
            pltpu.SemaphoreType.DMA,
            pltpu.SemaphoreType.DMA,
            pltpu.SemaphoreType.DMA,
            pltpu.SemaphoreType.DMA,
            pltpu.SemaphoreType.DMA,
            pltpu.SemaphoreType.DMA,
        ],
        compiler_params=cp,
    )
    return kern(h2f, a_s, a_d, src, dst)


def _gat_layer(h2, a_s, a_d, src, dst, bias, use_elu):
    num, den = _sc_agg(h2, a_s.reshape(N), a_d.reshape(N), src, dst)
    num = num.reshape(NCORE, N, DH)
    den = den.reshape(N, 1)
    return _dense_out(num, den, h2, a_s, a_d, bias, use_elu)


def kernel(x, spatial_edge_index, gene_sim_edge_index,
           W_sp, a_src_sp, a_dst_sp, b_sp,
           W_gs, a_src_gs, a_dst_gs, b_gs,
           W_f, a_src_f, a_dst_f, b_f):
    sp_src = spatial_edge_index[0]
    sp_dst = spatial_edge_index[1]
    gs_src = gene_sim_edge_index[0]
    gs_dst = gene_sim_edge_index[1]

    h2_sp, as_sp, ad_sp = _dense_in(x, W_sp, a_src_sp, a_dst_sp)
    h2_gs, as_gs, ad_gs = _dense_in(x, W_gs, a_src_gs, a_dst_gs)

    o1 = _gat_layer(h2_sp, as_sp, ad_sp, sp_src, sp_dst, b_sp, True)
    o2 = _gat_layer(h2_gs, as_gs, ad_gs, gs_src, gs_dst, b_gs, True)

    h2_f, as_f, ad_f = _dense_in_fused(o1, o2, W_f, a_src_f, a_dst_f)
    return _gat_layer(h2_f, as_f, ad_f, sp_src, sp_dst, b_f, False)
